# baseline (speedup 1.0000x reference)
import functools

import jax
import jax.numpy as jnp
from jax import lax
from jax.experimental import pallas as pl
from jax.experimental.pallas import tpu as pltpu
from jax.experimental.pallas import tpu_sc as plsc

B = 16384
F = 26
V = 100000
H = 128
L = 64

NW = 32
RSUB = 4
GSZ = RSUB * F
NLANE = 16
NBUF = 4
FLUSH_SUB = 16
FLUSH_ROWS = FLUSH_SUB * RSUB
ORING_GROUPS = 4
ORING_ROWS = ORING_GROUPS * FLUSH_ROWS


OFFP = 13


def _gather_sum_body(nsub, rows_per_w, idx_hbm, offp_hbm, tables_hbm, out_hbm,
                     idx_v, offv, buf0, buf1, buf2, buf3, out_v,
                     sem0, sem1, sem2, sem3, osem):
    NSUB = nsub
    c = lax.axis_index("c")
    s = lax.axis_index("s")
    wid = s * 2 + c

    bufs = (buf0, buf1, buf2, buf3)
    sems = (sem0, sem1, sem2, sem3)

    nwords = nsub * GSZ

    def add_offsets(vec0, nvec):
        def grp(gi, carry):
            w0 = (vec0 + gi * OFFP) * NLANE
            for k2 in range(OFFP):
                sl = pl.ds(w0 + k2 * NLANE, NLANE)
                idx_v[sl] = idx_v[sl] + offv[pl.ds(k2 * NLANE, NLANE)]
            return carry
        lax.fori_loop(0, nvec // OFFP, grp, 0)

    PRE = 8
    pltpu.sync_copy(offp_hbm, offv)
    pltpu.sync_copy(idx_hbm.at[pl.ds(wid * nwords, PRE * GSZ)],
                    idx_v.at[pl.ds(0, PRE * GSZ)])
    add_offsets(0, PRE * GSZ // NLANE)
    for p in range(NBUF - 1):
        pltpu.async_copy(tables_hbm.at[idx_v.at[pl.ds(p * GSZ, GSZ)]],
                         bufs[p], sems[p])
    pltpu.sync_copy(idx_hbm.at[pl.ds(wid * nwords + PRE * GSZ,
                                     (nsub - PRE) * GSZ)],
                    idx_v.at[pl.ds(PRE * GSZ, (nsub - PRE) * GSZ)])
    add_offsets(PRE * GSZ // NLANE, (nsub - PRE) * GSZ // NLANE)

    def accum(g, buf):
        def row_body(r, carry):
            orow = (g % (ORING_GROUPS * FLUSH_SUB)) * RSUB + r
            for j in range(H // NLANE):
                sl = pl.ds(j * NLANE, NLANE)
                vals = [buf[r * F + f, sl] for f in range(F)]
                while len(vals) > 1:
                    nxt = [vals[i] + vals[i + 1]
                           for i in range(0, len(vals) - 1, 2)]
                    if len(vals) % 2:
                        nxt.append(vals[-1])
                    vals = nxt
                out_v[orow, sl] = vals[0]
            return carry
        lax.fori_loop(0, RSUB, row_body, 0)

    def flush_wait():
        pltpu.make_async_copy(
            out_v.at[pl.ds(0, FLUSH_ROWS)],
            out_hbm.at[pl.ds(wid * rows_per_w, FLUSH_ROWS)],
            osem,
        ).wait()

    def outer(i, carry):
        for b in range(NBUF):
            g = i * NBUF + b
            buf, sem = bufs[b], sems[b]
            nb_i = (b + NBUF - 1) % NBUF
            nbuf, nsem = bufs[nb_i], sems[nb_i]

            @pl.when(g + NBUF - 1 < NSUB)
            def _issue():
                pltpu.async_copy(
                    tables_hbm.at[idx_v.at[pl.ds((g + NBUF - 1) * GSZ, GSZ)]],
                    nbuf, nsem)

            pltpu.make_async_copy(
                tables_hbm.at[idx_v.at[pl.ds(g * GSZ, GSZ)]], buf, sem).wait()
            accum(g, buf)

            @pl.when(g % FLUSH_SUB == FLUSH_SUB - 1)
            def _flush():
                grp = g // FLUSH_SUB
                ring0 = pl.multiple_of(
                    (grp % ORING_GROUPS) * FLUSH_ROWS, FLUSH_ROWS)
                hbm0 = pl.multiple_of(
                    wid * rows_per_w + (g - (FLUSH_SUB - 1)) * RSUB,
                    FLUSH_ROWS)
                pltpu.async_copy(
                    out_v.at[pl.ds(ring0, FLUSH_ROWS)],
                    out_hbm.at[pl.ds(hbm0, FLUSH_ROWS)],
                    osem,
                )
                @pl.when(grp >= ORING_GROUPS - 1)
                def _retire():
                    flush_wait()
        return carry

    lax.fori_loop(0, NSUB // NBUF, outer, 0)

    def drain(k, carry):
        flush_wait()
        return carry

    lax.fori_loop(0, ORING_GROUPS - 1, drain, 0)


@functools.lru_cache(maxsize=None)
def _make_gather_sum(nb):
    rows_per_w = nb // NW
    nsub = rows_per_w // RSUB
    mesh = plsc.VectorSubcoreMesh(core_axis_name="c", subcore_axis_name="s")
    return pl.kernel(
        functools.partial(_gather_sum_body, nsub, rows_per_w),
        out_type=jax.ShapeDtypeStruct((nb, H), jnp.float32),
        mesh=mesh,
        scratch_types=[
            pltpu.VMEM((nsub * GSZ,), jnp.int32),
            pltpu.VMEM((OFFP * NLANE,), jnp.int32),
            pltpu.VMEM((GSZ, H), jnp.float32),
            pltpu.VMEM((GSZ, H), jnp.float32),
            pltpu.VMEM((GSZ, H), jnp.float32),
            pltpu.VMEM((GSZ, H), jnp.float32),
            pltpu.VMEM((ORING_ROWS, H), jnp.float32),
            pltpu.SemaphoreType.DMA,
            pltpu.SemaphoreType.DMA,
            pltpu.SemaphoreType.DMA,
            pltpu.SemaphoreType.DMA,
            pltpu.SemaphoreType.DMA,
        ],
    )


def _gather_sum(idx1, offp, tables2d):
    nb = idx1.shape[0] // F
    return _make_gather_sum(nb)(idx1, offp, tables2d)


def _mlp_body(h_ref, w1_ref, b1_ref, wo_ref, bo_ref, mu_ref, lv_ref):
    h = h_ref[...]
    z = jnp.dot(h, w1_ref[...], preferred_element_type=jnp.float32)
    z = jnp.maximum(z + b1_ref[...], 0.0)
    z2 = (
        jnp.dot(z, wo_ref[...], preferred_element_type=jnp.float32)
        + bo_ref[...]
    )
    mu_ref[...] = z2[:, :L]
    lv_ref[...] = z2[:, L:]


@jax.jit
def _mlp(h, W1, b1, Wo, bo):
    TB = 2048
    nb = h.shape[0]
    grid = (nb // TB,)
    return pl.pallas_call(
        _mlp_body,
        grid=grid,
        in_specs=[
            pl.BlockSpec((TB, H), lambda i: (i, 0)),
            pl.BlockSpec((H, H), lambda i: (0, 0)),
            pl.BlockSpec((1, H), lambda i: (0, 0)),
            pl.BlockSpec((H, 2 * L), lambda i: (0, 0)),
            pl.BlockSpec((1, 2 * L), lambda i: (0, 0)),
        ],
        out_specs=[
            pl.BlockSpec((TB, L), lambda i: (i, 0)),
            pl.BlockSpec((TB, L), lambda i: (i, 0)),
        ],
        out_shape=[
            jax.ShapeDtypeStruct((nb, L), jnp.float32),
            jax.ShapeDtypeStruct((nb, L), jnp.float32),
        ],
    )(h, W1, b1, Wo, bo)


def kernel(x, tables, W1, b1, Wmu, bmu, Wvar, bvar):
    tables2d = tables.reshape(F * V, H)
    idx1 = x.reshape(B * F)
    offp = (jnp.arange(OFFP * NLANE, dtype=jnp.int32) % F) * V
    Wo = jnp.concatenate([Wmu, Wvar], axis=1)
    bo = jnp.concatenate([bmu, bvar]).reshape(1, 2 * L)
    h = _gather_sum(idx1, offp, tables2d)
    mu, lv = _mlp(h, W1, b1.reshape(1, H), Wo, bo)
    return mu, lv


# speedup vs baseline: 1.0268x; 1.0268x over previous
import functools

import jax
import jax.numpy as jnp
from jax import lax
from jax.experimental import pallas as pl
from jax.experimental.pallas import tpu as pltpu
from jax.experimental.pallas import tpu_sc as plsc

B = 16384
F = 26
V = 100000
H = 128
L = 64

NW = 32
RSUB = 8
GSZ = RSUB * F
NLANE = 16
NBUF = 3
FLUSH_SUB = 8
FLUSH_ROWS = FLUSH_SUB * RSUB
ORING_GROUPS = 4
ORING_ROWS = ORING_GROUPS * FLUSH_ROWS


def _gather_sum_body(nsub, rows_per_w, idx_hbm, tables_hbm, out_hbm, idx_v,
                     buf0, buf1, buf2, out_v, sem0, sem1, sem2, osem):
    NSUB = nsub
    c = lax.axis_index("c")
    s = lax.axis_index("s")
    wid = s * 2 + c

    bufs = (buf0, buf1, buf2)
    sems = (sem0, sem1, sem2)

    nwords = nsub * GSZ

    PRE = (NBUF - 1) * GSZ
    pltpu.sync_copy(idx_hbm.at[pl.ds(wid * nwords, PRE)],
                    idx_v.at[pl.ds(0, PRE)])
    for p in range(NBUF - 1):
        pltpu.async_copy(tables_hbm.at[idx_v.at[pl.ds(p * GSZ, GSZ)]],
                         bufs[p], sems[p])
    pltpu.sync_copy(idx_hbm.at[pl.ds(wid * nwords + PRE, nwords - PRE)],
                    idx_v.at[pl.ds(PRE, nwords - PRE)])

    def accum(g, buf):
        def row_body(r, carry):
            orow = (g % (ORING_GROUPS * FLUSH_SUB)) * RSUB + r
            for j in range(H // NLANE):
                sl = pl.ds(j * NLANE, NLANE)
                vals = [buf[r * F + f, sl] for f in range(F)]
                while len(vals) > 1:
                    nxt = [vals[i] + vals[i + 1]
                           for i in range(0, len(vals) - 1, 2)]
                    if len(vals) % 2:
                        nxt.append(vals[-1])
                    vals = nxt
                out_v[orow, sl] = vals[0]
            return carry
        lax.fori_loop(0, RSUB, row_body, 0)

    def flush_wait():
        pltpu.make_async_copy(
            out_v.at[pl.ds(0, FLUSH_ROWS)],
            out_hbm.at[pl.ds(wid * rows_per_w, FLUSH_ROWS)],
            osem,
        ).wait()

    def step(g, buf, sem, nbuf, nsem):
        @pl.when(g + NBUF - 1 < NSUB)
        def _issue():
            pltpu.async_copy(
                tables_hbm.at[idx_v.at[pl.ds((g + NBUF - 1) * GSZ, GSZ)]],
                nbuf, nsem)

        pltpu.make_async_copy(
            tables_hbm.at[idx_v.at[pl.ds(g * GSZ, GSZ)]], buf, sem).wait()
        accum(g, buf)

        @pl.when(g % FLUSH_SUB == FLUSH_SUB - 1)
        def _flush():
            grp = g // FLUSH_SUB
            ring0 = pl.multiple_of(
                (grp % ORING_GROUPS) * FLUSH_ROWS, FLUSH_ROWS)
            hbm0 = pl.multiple_of(
                wid * rows_per_w + (g - (FLUSH_SUB - 1)) * RSUB,
                FLUSH_ROWS)
            pltpu.async_copy(
                out_v.at[pl.ds(ring0, FLUSH_ROWS)],
                out_hbm.at[pl.ds(hbm0, FLUSH_ROWS)],
                osem,
            )
            @pl.when(grp >= ORING_GROUPS - 1)
            def _retire():
                flush_wait()

    NMAIN = (NSUB // NBUF) * NBUF

    def outer(i, carry):
        for b in range(NBUF):
            g = i * NBUF + b
            nb_i = (b + NBUF - 1) % NBUF
            step(g, bufs[b], sems[b], bufs[nb_i], sems[nb_i])
        return carry

    lax.fori_loop(0, NSUB // NBUF, outer, 0)

    for g in range(NMAIN, NSUB):
        b = g % NBUF
        nb_i = (b + NBUF - 1) % NBUF
        step(g, bufs[b], sems[b], bufs[nb_i], sems[nb_i])

    def drain(k, carry):
        flush_wait()
        return carry

    lax.fori_loop(0, ORING_GROUPS - 1, drain, 0)


@functools.lru_cache(maxsize=None)
def _make_gather_sum(nb):
    rows_per_w = nb // NW
    nsub = rows_per_w // RSUB
    mesh = plsc.VectorSubcoreMesh(core_axis_name="c", subcore_axis_name="s")
    return pl.kernel(
        functools.partial(_gather_sum_body, nsub, rows_per_w),
        out_type=jax.ShapeDtypeStruct((nb, H), jnp.float32),
        mesh=mesh,
        scratch_types=[
            pltpu.VMEM((nsub * GSZ,), jnp.int32),
            pltpu.VMEM((GSZ, H), jnp.float32),
            pltpu.VMEM((GSZ, H), jnp.float32),
            pltpu.VMEM((GSZ, H), jnp.float32),
            pltpu.VMEM((ORING_ROWS, H), jnp.float32),
            pltpu.SemaphoreType.DMA,
            pltpu.SemaphoreType.DMA,
            pltpu.SemaphoreType.DMA,
            pltpu.SemaphoreType.DMA,
        ],
    )


def _gather_sum(idx2, tables2d):
    nb = idx2.shape[0] // F
    return _make_gather_sum(nb)(idx2, tables2d)


def _mlp_body(h_ref, w1_ref, b1_ref, wo_ref, bo_ref, mu_ref, lv_ref):
    h = h_ref[...]
    z = jnp.dot(h, w1_ref[...], preferred_element_type=jnp.float32)
    z = jnp.maximum(z + b1_ref[...], 0.0)
    z2 = (
        jnp.dot(z, wo_ref[...], preferred_element_type=jnp.float32)
        + bo_ref[...]
    )
    mu_ref[...] = z2[:, :L]
    lv_ref[...] = z2[:, L:]


@jax.jit
def _mlp(h, W1, b1, Wo, bo):
    TB = 2048
    nb = h.shape[0]
    grid = (nb // TB,)
    return pl.pallas_call(
        _mlp_body,
        grid=grid,
        in_specs=[
            pl.BlockSpec((TB, H), lambda i: (i, 0)),
            pl.BlockSpec((H, H), lambda i: (0, 0)),
            pl.BlockSpec((1, H), lambda i: (0, 0)),
            pl.BlockSpec((H, 2 * L), lambda i: (0, 0)),
            pl.BlockSpec((1, 2 * L), lambda i: (0, 0)),
        ],
        out_specs=[
            pl.BlockSpec((TB, L), lambda i: (i, 0)),
            pl.BlockSpec((TB, L), lambda i: (i, 0)),
        ],
        out_shape=[
            jax.ShapeDtypeStruct((nb, L), jnp.float32),
            jax.ShapeDtypeStruct((nb, L), jnp.float32),
        ],
    )(h, W1, b1, Wo, bo)


def kernel(x, tables, W1, b1, Wmu, bmu, Wvar, bvar):
    tables2d = tables.reshape(F * V, H)
    offs = jnp.arange(F, dtype=jnp.int32) * V
    idx2 = (x.astype(jnp.int32) + offs[None, :]).reshape(B * F)
    Wo = jnp.concatenate([Wmu, Wvar], axis=1)
    bo = jnp.concatenate([bmu, bvar]).reshape(1, 2 * L)
    h = _gather_sum(idx2, tables2d)
    mu, lv = _mlp(h, W1, b1.reshape(1, H), Wo, bo)
    return mu, lv
